# baseline (speedup 1.0000x reference)
import functools

import jax
import jax.numpy as jnp
from jax import lax
from jax.experimental import pallas as pl
from jax.experimental.pallas import tpu as pltpu
from jax.experimental.pallas import tpu_sc as plsc

N = 10000
D = 128
C = 41

NP = 10240
CP = 48
DW = 8
L = 128
NW = 32
NB = 80
EPAD = NW * NB * L
SLAB = NP // 16
NBC = NB * 2

_mesh = plsc.VectorSubcoreMesh(core_axis_name="c", subcore_axis_name="s")
_mesh1 = plsc.VectorSubcoreMesh(
    core_axis_name="c", subcore_axis_name="s", num_cores=1)
_sc_params = pltpu.CompilerParams(use_tc_tiling_on_sc=False)


@functools.partial(
    pl.kernel,
    out_type=jax.ShapeDtypeStruct((2 * NP, DW), jnp.float32),
    mesh=_mesh,
    scratch_types=[
        pltpu.VMEM((NB, L), jnp.int32),
        pltpu.VMEM((L, DW), jnp.float32),
        pltpu.VMEM_SHARED((NP, DW), jnp.float32),
        pltpu.SemaphoreType.DMA,
    ],
    compiler_params=_sc_params,
)
def _deg_kernel(e_hbm, ones_hbm, zeros_hbm, out_hbm, dst_v, ones_v, deg_sh,
                sem):
    cid = lax.axis_index("c")
    sid = lax.axis_index("s")
    wid = cid * 16 + sid

    pltpu.sync_copy(e_hbm.at[1, pl.ds(wid * NB, NB)], dst_v)
    pltpu.sync_copy(ones_hbm, ones_v)
    pltpu.sync_copy(zeros_hbm, deg_sh.at[pl.ds(sid * SLAB, SLAB)])
    plsc.subcore_barrier()

    @pl.loop(0, NB)
    def _(j):
        pltpu.async_copy(ones_v, deg_sh.at[dst_v.at[j]], sem, add=True)

    @pl.loop(0, NB)
    def _(j):
        pltpu.make_async_copy(ones_v, deg_sh.at[dst_v.at[j]], sem).wait()

    plsc.subcore_barrier()
    pltpu.sync_copy(
        deg_sh.at[pl.ds(sid * SLAB, SLAB)],
        out_hbm.at[pl.ds(cid * NP + sid * SLAB, SLAB)],
    )


@functools.partial(
    pl.kernel,
    out_type=jax.ShapeDtypeStruct((2 * NP, CP), jnp.float32),
    mesh=_mesh,
    scratch_types=[
        pltpu.VMEM((NB, L), jnp.int32),
        pltpu.VMEM((NB, L), jnp.int32),
        pltpu.VMEM((4, L, CP), jnp.float32),
        pltpu.VMEM_SHARED((NP, CP), jnp.float32),
        pltpu.VMEM_SHARED((NP, CP), jnp.float32),
        pltpu.SemaphoreType.DMA((4,)),
        pltpu.SemaphoreType.DMA((4,)),
    ],
    compiler_params=_sc_params,
)
def _agg_kernel(y_hbm, e_hbm, zeros_hbm, out_hbm,
                src_v, dst_v, rows_v, y_sh, agg_sh, gsem, ssem):
    cid = lax.axis_index("c")
    sid = lax.axis_index("s")
    wid = cid * 16 + sid
    K = 4

    pltpu.sync_copy(e_hbm.at[0, pl.ds(wid * NB, NB)], src_v)
    pltpu.sync_copy(e_hbm.at[1, pl.ds(wid * NB, NB)], dst_v)
    pltpu.sync_copy(y_hbm.at[pl.ds(sid * SLAB, SLAB)],
                    y_sh.at[pl.ds(sid * SLAB, SLAB)])
    pltpu.sync_copy(zeros_hbm, agg_sh.at[pl.ds(sid * SLAB, SLAB)])
    plsc.subcore_barrier()

    for b in range(K):
        pltpu.async_copy(y_sh.at[src_v.at[b]], rows_v.at[b], gsem.at[b])

    @pl.loop(0, NB)
    def _(j):
        b = j % K
        bp = (j + K - 1) % K
        @pl.when(j >= 1)
        def _():
            jm = j - 1
            pltpu.make_async_copy(
                rows_v.at[bp], agg_sh.at[dst_v.at[jm]], ssem.at[bp]).wait()

            @pl.when(j + K - 1 < NB)
            def _():
                pltpu.async_copy(
                    y_sh.at[src_v.at[j + K - 1]], rows_v.at[bp],
                    gsem.at[bp])

        pltpu.make_async_copy(
            y_sh.at[src_v.at[j]], rows_v.at[b], gsem.at[b]).wait()
        pltpu.async_copy(rows_v.at[b], agg_sh.at[dst_v.at[j]], ssem.at[b],
                         add=True)

    bl = (NB - 1) % K
    pltpu.make_async_copy(
        rows_v.at[bl], agg_sh.at[dst_v.at[NB - 1]], ssem.at[bl]).wait()
    plsc.subcore_barrier()
    pltpu.sync_copy(
        agg_sh.at[pl.ds(sid * SLAB, SLAB)],
        out_hbm.at[pl.ds(cid * NP + sid * SLAB, SLAB)],
    )


def _xw_body(x_ref, w_ref, d0_ref, d1_ref, y_ref, dinv_ref):
    deg = d0_ref[:, 0:1] + d1_ref[:, 0:1] + 1.0
    dinv = lax.rsqrt(deg)
    xw = jnp.dot(x_ref[...], w_ref[...], preferred_element_type=jnp.float32)
    y_ref[...] = dinv * xw
    dinv_ref[...] = dinv


_RB = 2048


def _run_xw(x, w_pad, deg0, deg1):
    return pl.pallas_call(
        _xw_body,
        grid=(NP // _RB,),
        in_specs=[
            pl.BlockSpec((_RB, D), lambda i: (i, 0)),
            pl.BlockSpec((D, CP), lambda i: (0, 0)),
            pl.BlockSpec((_RB, DW), lambda i: (i, 0)),
            pl.BlockSpec((_RB, DW), lambda i: (i, 0)),
        ],
        out_specs=[
            pl.BlockSpec((_RB, CP), lambda i: (i, 0)),
            pl.BlockSpec((_RB, 1), lambda i: (i, 0)),
        ],
        out_shape=[
            jax.ShapeDtypeStruct((NP, CP), jnp.float32),
            jax.ShapeDtypeStruct((NP, 1), jnp.float32),
        ],
    )(x, w_pad, deg0, deg1)


def _fin_body(a0_ref, a1_ref, y_ref, dinv_ref, b_ref, out_ref, z_ref):
    o = dinv_ref[...] * (a0_ref[...] + a1_ref[...] + y_ref[...]) + b_ref[...]
    col = lax.broadcasted_iota(jnp.int32, o.shape, 1)
    valid = col < C
    neg = jnp.float32(-1e30)
    m = jnp.max(jnp.where(valid, o, neg), axis=1, keepdims=True)
    s = jnp.sum(jnp.where(valid, jnp.exp(o - m), 0.0), axis=1, keepdims=True)
    z = o - (m + jnp.log(s))
    out_ref[...] = o[:, :C]
    z_ref[...] = z[:, :C]


_RF = 2000


def _run_fin(agg0, agg1, y, dinv, b_pad):
    return pl.pallas_call(
        _fin_body,
        grid=(N // _RF,),
        in_specs=[
            pl.BlockSpec((_RF, CP), lambda i: (i, 0)),
            pl.BlockSpec((_RF, CP), lambda i: (i, 0)),
            pl.BlockSpec((_RF, CP), lambda i: (i, 0)),
            pl.BlockSpec((_RF, 1), lambda i: (i, 0)),
            pl.BlockSpec((1, CP), lambda i: (0, 0)),
        ],
        out_specs=[
            pl.BlockSpec((_RF, C), lambda i: (i, 0)),
            pl.BlockSpec((_RF, C), lambda i: (i, 0)),
        ],
        out_shape=[
            jax.ShapeDtypeStruct((N, C), jnp.float32),
            jax.ShapeDtypeStruct((N, C), jnp.float32),
        ],
    )(agg0, agg1, y, dinv, b_pad)


def kernel(x, edge_index, W, b):
    E = edge_index.shape[1]
    e3 = jnp.pad(
        edge_index.astype(jnp.int32), ((0, 0), (0, EPAD - E)),
        constant_values=N,
    ).reshape(2, NW * NB, L)

    w_pad = jnp.zeros((D, CP), jnp.float32).at[:, :C].set(W.astype(jnp.float32))
    b_pad = jnp.zeros((1, CP), jnp.float32).at[0, :C].set(b.astype(jnp.float32))

    ones_dw = jnp.ones((L, DW), jnp.float32)
    zeros_dw = jnp.zeros((SLAB, DW), jnp.float32)
    zeros_cp = jnp.zeros((SLAB, CP), jnp.float32)

    x_pad = jnp.zeros((NP, D), jnp.float32).at[:N].set(x.astype(jnp.float32))
    degp = _deg_kernel(e3, ones_dw, zeros_dw)
    y, dinv = _run_xw(x_pad, w_pad, degp[:NP], degp[NP:])
    aggp = _agg_kernel(y, e3, zeros_cp)
    out, z = _run_fin(aggp[:NP], aggp[NP:], y, dinv, b_pad)
    return (out, z)


# speedup vs baseline: 1.1100x; 1.1100x over previous
import functools

import jax
import jax.numpy as jnp
from jax import lax
from jax.experimental import pallas as pl
from jax.experimental.pallas import tpu as pltpu
from jax.experimental.pallas import tpu_sc as plsc

N = 10000
D = 128
C = 41

NP = 10240
CP = 48
DW = 8
L = 128
NW = 32
NB = 80
EPAD = NW * NB * L
SLAB = NP // 16
NBC = NB * 2

_mesh = plsc.VectorSubcoreMesh(core_axis_name="c", subcore_axis_name="s")
_mesh1 = plsc.VectorSubcoreMesh(
    core_axis_name="c", subcore_axis_name="s", num_cores=1)
_sc_params = pltpu.CompilerParams(use_tc_tiling_on_sc=False, needs_layout_passes=False)


@functools.partial(
    pl.kernel,
    out_type=jax.ShapeDtypeStruct((2 * NP, DW), jnp.float32),
    mesh=_mesh,
    scratch_types=[
        pltpu.VMEM((NB, L), jnp.int32),
        pltpu.VMEM((L, DW), jnp.float32),
        pltpu.VMEM_SHARED((NP, DW), jnp.float32),
        pltpu.SemaphoreType.DMA,
    ],
    compiler_params=_sc_params,
)
def _deg_kernel(e_hbm, ones_hbm, zeros_hbm, out_hbm, dst_v, ones_v, deg_sh,
                sem):
    cid = lax.axis_index("c")
    sid = lax.axis_index("s")
    wid = cid * 16 + sid

    pltpu.sync_copy(e_hbm.at[1, pl.ds(wid * NB, NB)], dst_v)
    pltpu.sync_copy(ones_hbm, ones_v)
    pltpu.sync_copy(zeros_hbm, deg_sh.at[pl.ds(sid * SLAB, SLAB)])
    plsc.subcore_barrier()

    @pl.loop(0, NB)
    def _(j):
        pltpu.async_copy(ones_v, deg_sh.at[dst_v.at[j]], sem, add=True)

    @pl.loop(0, NB)
    def _(j):
        pltpu.make_async_copy(ones_v, deg_sh.at[dst_v.at[j]], sem).wait()

    plsc.subcore_barrier()
    pltpu.sync_copy(
        deg_sh.at[pl.ds(sid * SLAB, SLAB)],
        out_hbm.at[pl.ds(cid * NP + sid * SLAB, SLAB)],
    )


CSZ = SLAB // 2


@functools.partial(
    pl.kernel,
    out_type=jax.ShapeDtypeStruct((2 * NP, CP), jnp.float32),
    mesh=_mesh,
    scratch_types=[
        pltpu.VMEM((NB, L), jnp.int32),
        pltpu.VMEM((NB, L), jnp.int32),
        pltpu.VMEM((2, L, CP), jnp.float32),
        pltpu.VMEM((CSZ, CP), jnp.float32),
        pltpu.VMEM((CSZ * DW // 16, 16), jnp.float32),
        pltpu.VMEM((CSZ * DW // 16, 16), jnp.float32),
        pltpu.VMEM((SLAB * DW,), jnp.float32),
        pltpu.VMEM_SHARED((NP, CP), jnp.float32),
        pltpu.VMEM_SHARED((NP, CP), jnp.float32),
        pltpu.SemaphoreType.DMA((2,)),
        pltpu.SemaphoreType.DMA((2,)),
    ],
    compiler_params=_sc_params,
)
def _agg_kernel(xw_hbm, deg_hbm, e_hbm, zeros_hbm, out_hbm,
                src_v, dst_v, rows_v, xw_v, d0_v, d1_v, dinv_v,
                y_sh, agg_sh, gsem, ssem):
    cid = lax.axis_index("c")
    sid = lax.axis_index("s")
    wid = cid * 16 + sid
    K = 2
    NV = CSZ * DW // 16

    pltpu.sync_copy(e_hbm.at[0, pl.ds(wid * NB, NB)], src_v)
    pltpu.sync_copy(e_hbm.at[1, pl.ds(wid * NB, NB)], dst_v)

    for q in range(2):
        base = sid * SLAB + q * CSZ
        pltpu.sync_copy(deg_hbm.at[pl.ds(base * DW // 16, NV)], d0_v)
        pltpu.sync_copy(deg_hbm.at[pl.ds((NP + base) * DW // 16, NV)], d1_v)
        pltpu.sync_copy(xw_hbm.at[pl.ds(base, CSZ)], xw_v)

        @pl.loop(0, NV)
        def _(v):
            d = d0_v[v] + d1_v[v] + 1.0
            i = plsc.bitcast(d, jnp.int32)
            yk = plsc.bitcast(jnp.int32(0x5F3759DF) - (i >> 1), jnp.float32)
            for _ in range(3):
                yk = yk * (1.5 - 0.5 * d * yk * yk)
            dinv_v[pl.ds(q * CSZ * DW + v * 16, 16)] = yk

        @pl.loop(0, CSZ)
        def _(r):
            idx = (lax.iota(jnp.int32, 16) * 0
                   + (q * CSZ + r) * DW).astype(jnp.int32)
            yk = plsc.load_gather(dinv_v, [idx])
            for c in range(3):
                xw_v[r, pl.ds(16 * c, 16)] = xw_v[r, pl.ds(16 * c, 16)] * yk

        pltpu.sync_copy(xw_v, y_sh.at[pl.ds(base, CSZ)])

        @pl.when(cid == 0)
        def _():
            pltpu.sync_copy(xw_v, agg_sh.at[pl.ds(base, CSZ)])

        @pl.when(cid != 0)
        def _():
            pltpu.sync_copy(zeros_hbm.at[pl.ds(q * CSZ, CSZ)],
                            agg_sh.at[pl.ds(base, CSZ)])

    plsc.subcore_barrier()

    for b in range(K):
        pltpu.async_copy(y_sh.at[src_v.at[b]], rows_v.at[b], gsem.at[b])

    @pl.loop(0, NB)
    def _(j):
        b = j % K
        bp = (j + K - 1) % K
        @pl.when(j >= 1)
        def _():
            jm = j - 1
            pltpu.make_async_copy(
                rows_v.at[bp], agg_sh.at[dst_v.at[jm]], ssem.at[bp]).wait()

            @pl.when(j + K - 1 < NB)
            def _():
                pltpu.async_copy(
                    y_sh.at[src_v.at[j + K - 1]], rows_v.at[bp],
                    gsem.at[bp])

        pltpu.make_async_copy(
            y_sh.at[src_v.at[j]], rows_v.at[b], gsem.at[b]).wait()
        pltpu.async_copy(rows_v.at[b], agg_sh.at[dst_v.at[j]], ssem.at[b],
                         add=True)

    bl = (NB - 1) % K
    pltpu.make_async_copy(
        rows_v.at[bl], agg_sh.at[dst_v.at[NB - 1]], ssem.at[bl]).wait()
    plsc.subcore_barrier()

    for q in range(2):
        base = sid * SLAB + q * CSZ
        pltpu.sync_copy(agg_sh.at[pl.ds(base, CSZ)], xw_v)

        @pl.loop(0, CSZ)
        def _(r):
            idx = (lax.iota(jnp.int32, 16) * 0
                   + (q * CSZ + r) * DW).astype(jnp.int32)
            yk = plsc.load_gather(dinv_v, [idx])
            for c in range(3):
                xw_v[r, pl.ds(16 * c, 16)] = xw_v[r, pl.ds(16 * c, 16)] * yk

        pltpu.sync_copy(xw_v, out_hbm.at[pl.ds(cid * NP + base, CSZ)])


def _xw_body(x_ref, w_ref, xw_ref):
    xw_ref[...] = jnp.dot(
        x_ref[...], w_ref[...], preferred_element_type=jnp.float32)


_RB = 2048


def _run_xw(x, w_pad):
    return pl.pallas_call(
        _xw_body,
        grid=(NP // _RB,),
        in_specs=[
            pl.BlockSpec((_RB, D), lambda i: (i, 0)),
            pl.BlockSpec((D, CP), lambda i: (0, 0)),
        ],
        out_specs=pl.BlockSpec((_RB, CP), lambda i: (i, 0)),
        out_shape=jax.ShapeDtypeStruct((NP, CP), jnp.float32),
    )(x, w_pad)


def _fin_body(a0_ref, a1_ref, b_ref, out_ref, z_ref):
    o = a0_ref[...] + a1_ref[...] + b_ref[...]
    col = lax.broadcasted_iota(jnp.int32, o.shape, 1)
    valid = col < C
    neg = jnp.float32(-1e30)
    m = jnp.max(jnp.where(valid, o, neg), axis=1, keepdims=True)
    s = jnp.sum(jnp.where(valid, jnp.exp(o - m), 0.0), axis=1, keepdims=True)
    z = o - (m + jnp.log(s))
    out_ref[...] = o[:, :C]
    z_ref[...] = z[:, :C]


_RF = 2000


def _run_fin(agg0, agg1, b_pad):
    return pl.pallas_call(
        _fin_body,
        grid=(N // _RF,),
        in_specs=[
            pl.BlockSpec((_RF, CP), lambda i: (i, 0)),
            pl.BlockSpec((_RF, CP), lambda i: (i, 0)),
            pl.BlockSpec((1, CP), lambda i: (0, 0)),
        ],
        out_specs=[
            pl.BlockSpec((_RF, C), lambda i: (i, 0)),
            pl.BlockSpec((_RF, C), lambda i: (i, 0)),
        ],
        out_shape=[
            jax.ShapeDtypeStruct((N, C), jnp.float32),
            jax.ShapeDtypeStruct((N, C), jnp.float32),
        ],
    )(agg0, agg1, b_pad)


def kernel(x, edge_index, W, b):
    E = edge_index.shape[1]
    e3 = jnp.pad(
        edge_index.astype(jnp.int32), ((0, 0), (0, EPAD - E)),
        constant_values=N,
    ).reshape(2, NW * NB, L)

    w_pad = jnp.zeros((D, CP), jnp.float32).at[:, :C].set(W.astype(jnp.float32))
    b_pad = jnp.zeros((1, CP), jnp.float32).at[0, :C].set(b.astype(jnp.float32))

    ones_dw = jnp.ones((L, DW), jnp.float32)
    zeros_dw = jnp.zeros((SLAB, DW), jnp.float32)
    zeros_cp = jnp.zeros((SLAB, CP), jnp.float32)

    x_pad = jnp.zeros((NP, D), jnp.float32).at[:N].set(x.astype(jnp.float32))
    degp = _deg_kernel(e3, ones_dw, zeros_dw)
    xw = _run_xw(x_pad, w_pad)
    aggp = _agg_kernel(xw, degp.reshape(NP, 16), e3, zeros_cp)
    out, z = _run_fin(aggp[:NP], aggp[NP:], b_pad)
    return (out, z)


# speedup vs baseline: 1.1632x; 1.0480x over previous
import functools

import jax
import jax.numpy as jnp
from jax import lax
from jax.experimental import pallas as pl
from jax.experimental.pallas import tpu as pltpu
from jax.experimental.pallas import tpu_sc as plsc

N = 10000
D = 128
C = 41

NP = 10240
CP = 48
DW = 8
L = 128
NW = 32
NB = 80
EPAD = NW * NB * L
SLAB = NP // 16
NBC = NB * 2

_mesh = plsc.VectorSubcoreMesh(core_axis_name="c", subcore_axis_name="s")
_mesh1 = plsc.VectorSubcoreMesh(
    core_axis_name="c", subcore_axis_name="s", num_cores=1)
_sc_params = pltpu.CompilerParams(use_tc_tiling_on_sc=False, needs_layout_passes=False)


@functools.partial(
    pl.kernel,
    out_type=jax.ShapeDtypeStruct((2 * NP, DW), jnp.float32),
    mesh=_mesh,
    scratch_types=[
        pltpu.VMEM((NB, L), jnp.int32),
        pltpu.VMEM((L, DW), jnp.float32),
        pltpu.VMEM_SHARED((NP, DW), jnp.float32),
        pltpu.SemaphoreType.DMA,
    ],
    compiler_params=_sc_params,
)
def _deg_kernel(e_hbm, ones_hbm, zeros_hbm, out_hbm, dst_v, ones_v, deg_sh,
                sem):
    cid = lax.axis_index("c")
    sid = lax.axis_index("s")
    wid = cid * 16 + sid

    pltpu.sync_copy(e_hbm.at[1, pl.ds(wid * NB, NB)], dst_v)
    pltpu.sync_copy(ones_hbm, ones_v)
    pltpu.sync_copy(zeros_hbm, deg_sh.at[pl.ds(sid * SLAB, SLAB)])
    plsc.subcore_barrier()

    @pl.loop(0, NB)
    def _(j):
        pltpu.async_copy(ones_v, deg_sh.at[dst_v.at[j]], sem, add=True)

    @pl.loop(0, NB)
    def _(j):
        pltpu.make_async_copy(ones_v, deg_sh.at[dst_v.at[j]], sem).wait()

    plsc.subcore_barrier()
    pltpu.sync_copy(
        deg_sh.at[pl.ds(sid * SLAB, SLAB)],
        out_hbm.at[pl.ds(cid * NP + sid * SLAB, SLAB)],
    )


CSZ = SLAB // 2


@functools.partial(
    pl.kernel,
    out_type=jax.ShapeDtypeStruct((2 * NP, CP), jnp.float32),
    mesh=_mesh,
    scratch_types=[
        pltpu.VMEM((NB, L), jnp.int32),
        pltpu.VMEM((NB, L), jnp.int32),
        pltpu.VMEM((2, L, CP), jnp.float32),
        pltpu.VMEM((CSZ, CP), jnp.float32),
        pltpu.VMEM((CSZ * DW // 16, 16), jnp.float32),
        pltpu.VMEM((CSZ * DW // 16, 16), jnp.float32),
        pltpu.VMEM((SLAB * DW,), jnp.float32),
        pltpu.VMEM_SHARED((NP, CP), jnp.float32),
        pltpu.VMEM_SHARED((NP, CP), jnp.float32),
        pltpu.SemaphoreType.DMA((2,)),
        pltpu.SemaphoreType.DMA((2,)),
    ],
    compiler_params=_sc_params,
)
def _agg_kernel(xw_hbm, deg_hbm, e_hbm, zeros_hbm, out_hbm,
                src_v, dst_v, rows_v, xw_v, d0_v, d1_v, dinv_v,
                y_sh, agg_sh, gsem, ssem):
    cid = lax.axis_index("c")
    sid = lax.axis_index("s")
    wid = cid * 16 + sid
    K = 2
    NV = CSZ * DW // 16

    pltpu.sync_copy(e_hbm.at[0, pl.ds(wid * NB, NB)], src_v)
    pltpu.sync_copy(e_hbm.at[1, pl.ds(wid * NB, NB)], dst_v)

    for q in range(2):
        base = sid * SLAB + q * CSZ
        pltpu.sync_copy(deg_hbm.at[pl.ds(base * DW // 16, NV)], d0_v)
        pltpu.sync_copy(deg_hbm.at[pl.ds((NP + base) * DW // 16, NV)], d1_v)
        pltpu.sync_copy(xw_hbm.at[pl.ds(base, CSZ)], xw_v)

        @pl.loop(0, NV)
        def _(v):
            d = d0_v[v] + d1_v[v] + 1.0
            i = plsc.bitcast(d, jnp.int32)
            yk = plsc.bitcast(jnp.int32(0x5F3759DF) - (i >> 1), jnp.float32)
            for _ in range(3):
                yk = yk * (1.5 - 0.5 * d * yk * yk)
            dinv_v[pl.ds(q * CSZ * DW + v * 16, 16)] = yk

        @pl.loop(0, CSZ)
        def _(r):
            idx = (lax.iota(jnp.int32, 16) * 0
                   + (q * CSZ + r) * DW).astype(jnp.int32)
            yk = plsc.load_gather(dinv_v, [idx])
            for c in range(3):
                xw_v[r, pl.ds(16 * c, 16)] = xw_v[r, pl.ds(16 * c, 16)] * yk

        pltpu.sync_copy(xw_v, y_sh.at[pl.ds(base, CSZ)])

        @pl.when(cid == 0)
        def _():
            pltpu.sync_copy(xw_v, agg_sh.at[pl.ds(base, CSZ)])

        @pl.when(cid != 0)
        def _():
            pltpu.sync_copy(zeros_hbm.at[pl.ds(q * CSZ, CSZ)],
                            agg_sh.at[pl.ds(base, CSZ)])

    plsc.subcore_barrier()

    for b in range(K):
        pltpu.async_copy(y_sh.at[src_v.at[b]], rows_v.at[b], gsem.at[b])

    @pl.loop(0, NB)
    def _(j):
        b = j % K
        bp = (j + K - 1) % K
        @pl.when(j >= 1)
        def _():
            jm = j - 1
            pltpu.make_async_copy(
                rows_v.at[bp], agg_sh.at[dst_v.at[jm]], ssem.at[bp]).wait()

            @pl.when(j + K - 1 < NB)
            def _():
                pltpu.async_copy(
                    y_sh.at[src_v.at[j + K - 1]], rows_v.at[bp],
                    gsem.at[bp])

        pltpu.make_async_copy(
            y_sh.at[src_v.at[j]], rows_v.at[b], gsem.at[b]).wait()
        pltpu.async_copy(rows_v.at[b], agg_sh.at[dst_v.at[j]], ssem.at[b],
                         add=True)

    bl = (NB - 1) % K
    pltpu.make_async_copy(
        rows_v.at[bl], agg_sh.at[dst_v.at[NB - 1]], ssem.at[bl]).wait()
    plsc.subcore_barrier()

    for q in range(2):
        base = sid * SLAB + q * CSZ
        pltpu.sync_copy(agg_sh.at[pl.ds(base, CSZ)], xw_v)

        @pl.loop(0, CSZ)
        def _(r):
            idx = (lax.iota(jnp.int32, 16) * 0
                   + (q * CSZ + r) * DW).astype(jnp.int32)
            yk = plsc.load_gather(dinv_v, [idx])
            for c in range(3):
                xw_v[r, pl.ds(16 * c, 16)] = xw_v[r, pl.ds(16 * c, 16)] * yk

        pltpu.sync_copy(xw_v, out_hbm.at[pl.ds(cid * NP + base, CSZ)])


def _xw_body(x_ref, w_ref, xw_ref):
    xw_ref[...] = jnp.dot(
        x_ref[...], w_ref[...], preferred_element_type=jnp.float32)


_RB = 2048


def _run_xw(x, w_pad):
    return pl.pallas_call(
        _xw_body,
        grid=(NP // _RB,),
        in_specs=[
            pl.BlockSpec((_RB, D), lambda i: (i, 0)),
            pl.BlockSpec((D, CP), lambda i: (0, 0)),
        ],
        out_specs=pl.BlockSpec((_RB, CP), lambda i: (i, 0)),
        out_shape=jax.ShapeDtypeStruct((NP, CP), jnp.float32),
    )(x, w_pad)


def _fin_body(a0_ref, a1_ref, b_ref, out_ref, z_ref):
    o = a0_ref[0] + a1_ref[0] + b_ref[...]
    col = lax.broadcasted_iota(jnp.int32, o.shape, 1)
    valid = col < C
    neg = jnp.float32(-1e30)
    m = jnp.max(jnp.where(valid, o, neg), axis=1, keepdims=True)
    s = jnp.sum(jnp.where(valid, jnp.exp(o - m), 0.0), axis=1, keepdims=True)
    z = o - (m + jnp.log(s))
    out_ref[...] = o[:, :C]
    z_ref[...] = z[:, :C]


_RF = 2000


def _run_fin(aggp3, b_pad):
    return pl.pallas_call(
        _fin_body,
        grid=(N // _RF,),
        in_specs=[
            pl.BlockSpec((1, _RF, CP), lambda i: (0, i, 0)),
            pl.BlockSpec((1, _RF, CP), lambda i: (1, i, 0)),
            pl.BlockSpec((1, CP), lambda i: (0, 0)),
        ],
        out_specs=[
            pl.BlockSpec((_RF, C), lambda i: (i, 0)),
            pl.BlockSpec((_RF, C), lambda i: (i, 0)),
        ],
        out_shape=[
            jax.ShapeDtypeStruct((N, C), jnp.float32),
            jax.ShapeDtypeStruct((N, C), jnp.float32),
        ],
    )(aggp3, aggp3, b_pad)


def kernel(x, edge_index, W, b):
    E = edge_index.shape[1]
    e3 = jnp.pad(
        edge_index.astype(jnp.int32), ((0, 0), (0, EPAD - E)),
        constant_values=N,
    ).reshape(2, NW * NB, L)

    w_pad = jnp.zeros((D, CP), jnp.float32).at[:, :C].set(W.astype(jnp.float32))
    b_pad = jnp.zeros((1, CP), jnp.float32).at[0, :C].set(b.astype(jnp.float32))

    ones_dw = jnp.ones((L, DW), jnp.float32)
    zeros_dw = jnp.zeros((SLAB, DW), jnp.float32)
    zeros_cp = jnp.zeros((SLAB, CP), jnp.float32)

    x_pad = jnp.zeros((NP, D), jnp.float32).at[:N].set(x.astype(jnp.float32))
    degp = _deg_kernel(e3, ones_dw, zeros_dw)
    xw = _run_xw(x_pad, w_pad)
    aggp = _agg_kernel(xw, degp.reshape(NP, 16), e3, zeros_cp)
    out, z = _run_fin(aggp.reshape(2, NP, CP), b_pad)
    return (out, z)
